# baseline (speedup 1.0000x reference)
import jax
import jax.numpy as jnp
from jax.experimental import pallas as pl
from jax.experimental.pallas import tpu as pltpu

_N = 100
_IN = 128
_OUT = 128
_E = 16
_B_BLK = 128


def _combine_kernel(gw_ref, experts_ref, biases_ref, cw_ref, cb_ref):
    g = jax.nn.softmax(gw_ref[...], axis=-1)
    cw_ref[...] = jnp.dot(g, experts_ref[...], preferred_element_type=jnp.float32)
    cb_ref[...] = jnp.dot(g, biases_ref[...], preferred_element_type=jnp.float32)


def _apply_kernel(x_ref, cw_ref, cb_ref, out_ref):
    for n in range(_N):
        xn = x_ref[:, n, :]
        wn = cw_ref[n]
        yn = jnp.dot(xn, wn, preferred_element_type=jnp.float32)
        out_ref[:, n, :] = yn + cb_ref[n : n + 1, :]


def kernel(x, gate_weights, experts, expert_biases):
    batch = x.shape[0]
    experts2 = experts.reshape(_E, _IN * _OUT)

    cw2, cb = pl.pallas_call(
        _combine_kernel,
        out_shape=(
            jax.ShapeDtypeStruct((_N, _IN * _OUT), jnp.float32),
            jax.ShapeDtypeStruct((_N, _OUT), jnp.float32),
        ),
    )(gate_weights, experts2, expert_biases)
    cw = cw2.reshape(_N, _IN, _OUT)

    grid = (batch // _B_BLK,)
    out = pl.pallas_call(
        _apply_kernel,
        grid=grid,
        in_specs=[
            pl.BlockSpec((_B_BLK, _N, _IN), lambda i: (i, 0, 0)),
            pl.BlockSpec((_N, _IN, _OUT), lambda i: (0, 0, 0)),
            pl.BlockSpec((_N, _OUT), lambda i: (0, 0)),
        ],
        out_specs=pl.BlockSpec((_B_BLK, _N, _OUT), lambda i: (i, 0, 0)),
        out_shape=jax.ShapeDtypeStruct((batch, _N, _OUT), jnp.float32),
        compiler_params=pltpu.CompilerParams(
            dimension_semantics=("arbitrary",),
        ),
    )(x, cw, cb)
    return out
